# baseline (speedup 1.0000x reference)
import functools

import jax
import jax.numpy as jnp
from jax import lax
from jax.experimental import pallas as pl
from jax.experimental.pallas import tpu as pltpu
from jax.experimental.pallas import tpu_sc as plsc

B = 8
H = W = 112
C = 96
F = H * W * C
UPS = 2
OH = H * UPS
OW = W * UPS
PLANE = OW * C
S = OH * PLANE

NC = 2
NS = 16
L = 16

NCHUNK = 3
KP = 75
CHUNKA = KP * PLANE
CHUNKB = (OH - 2 * KP) * PLANE
DUMP = 2048
ACC = CHUNKA + DUMP
NTASK = B * NCHUNK

PER_TILE = F // NS
PIECE = 2688
NPIECE = PER_TILE // PIECE
PC = PIECE + 128
ACC_Z = ACC // NS
ZW = 5312
WB_A0 = 788 * 128
WB_A1 = 787 * 128
OUT_TB = CHUNKB // NS


def _body(feat_hbm, idx_hbm, out_hbm, idx_v, feat_v, adjc, featc, zero_v,
          acc_sh, sem_in, sem_sc, sem_z):
    core = lax.axis_index("c")
    tile = lax.axis_index("s")
    iota = lax.iota(jnp.int32, L)

    def _zinit(g, _):
        zero_v[pl.ds(g * L, L)] = jnp.zeros((L,), jnp.float32)
        return 0

    lax.fori_loop(0, ZW // L, _zinit, 0)

    def _stage_start(p, par, in_base):
        src = in_base + p * PIECE
        pltpu.async_copy(idx_hbm.at[pl.ds(src, PIECE)],
                         idx_v.at[pl.ds(par * PIECE, PIECE)], sem_in)
        pltpu.async_copy(feat_hbm.at[pl.ds(src, PIECE)],
                         feat_v.at[pl.ds(par * PIECE, PIECE)], sem_in)

    def _stage_wait(p, par, in_base):
        src = in_base + p * PIECE
        pltpu.make_async_copy(idx_hbm.at[pl.ds(src, PIECE)],
                              idx_v.at[pl.ds(par * PIECE, PIECE)],
                              sem_in).wait()
        pltpu.make_async_copy(feat_hbm.at[pl.ds(src, PIECE)],
                              feat_v.at[pl.ds(par * PIECE, PIECE)],
                              sem_in).wait()

    def _compact(par, base_k, chunk_size):
        def _grp(g, wptr):
            off = par * PIECE + g * L
            raw = idx_v[pl.ds(off, L)]
            local = raw - base_k
            ok = (local >= 0) & (local < chunk_size)
            vals = feat_v[pl.ds(off, L)]
            plsc.store_compressed(adjc.at[pl.ds(par * PC + wptr, L)], local,
                                  mask=ok)
            plsc.store_compressed(featc.at[pl.ds(par * PIECE + wptr, L)],
                                  vals, mask=ok)
            pcnt = plsc.all_reduce_population_count(ok)
            return wptr + pcnt[0]

        return lax.fori_loop(0, PIECE // L, _grp, jnp.int32(0))

    def _pad(par, cnt, p):
        for q in range(8):
            offs = (p * 256 + q * 16 + tile * 64) & (DUMP - 16)
            adjc[pl.ds(par * PC + cnt + q * L, L)] = CHUNKA + offs + iota

    def _scatter_fire(par, rows):
        def _row(j, _):
            pltpu.async_copy(
                featc.at[pl.ds(par * PIECE + j * 128, 128)],
                acc_sh.at[adjc.at[pl.ds(par * PC + j * 128, 128)]],
                sem_sc, add=True)
            return 0

        lax.fori_loop(0, rows, _row, 0)

    def _scatter_drain(par, rows):
        def _row(j, _):
            pltpu.make_async_copy(
                featc.at[pl.ds(par * PIECE + j * 128, 128)],
                acc_sh.at[adjc.at[pl.ds(par * PC + j * 128, 128)]],
                sem_sc).wait()
            return 0

        lax.fori_loop(0, rows, _row, 0)

    def _task(i, _):
        t = i * NC + core
        b = t // NCHUNK
        k = t % NCHUNK
        base_k = k * CHUNKA
        chunk_size = jnp.where(k == 2, CHUNKB, CHUNKA)
        in_base = b * F + tile * PER_TILE

        def _zfire(q, _):
            pltpu.async_copy(zero_v,
                             acc_sh.at[pl.ds(tile * ACC_Z + q * ZW, ZW)],
                             sem_z)
            return 0

        def _zdrain(q, _):
            pltpu.make_async_copy(
                zero_v, acc_sh.at[pl.ds(tile * ACC_Z + q * ZW, ZW)],
                sem_z).wait()
            return 0

        lax.fori_loop(0, ACC_Z // ZW, _zfire, 0)
        lax.fori_loop(0, ACC_Z // ZW, _zdrain, 0)
        plsc.subcore_barrier()

        _stage_start(0, 0, in_base)

        def _piece(p, rows_prev):
            cur = lax.rem(p, 2)
            nxt = 1 - cur
            _stage_wait(p, cur, in_base)
            cnt = _compact(cur, base_k, chunk_size)
            _pad(cur, cnt, p)
            rows = (cnt + 127) // 128

            @pl.when(p > 0)
            def _():
                _scatter_drain(nxt, rows_prev)

            _scatter_fire(cur, rows)

            @pl.when(p + 1 < NPIECE)
            def _():
                _stage_start(p + 1, nxt, in_base)

            return rows

        rows_last = lax.fori_loop(0, NPIECE, _piece, jnp.int32(0))
        _scatter_drain((NPIECE - 1) % 2, rows_last)
        plsc.subcore_barrier()

        @pl.when((k < 2) & (tile < 8))
        def _():
            off = tile * WB_A0
            pltpu.sync_copy(acc_sh.at[pl.ds(off, WB_A0)],
                            out_hbm.at[pl.ds(b * S + base_k + off, WB_A0)])

        @pl.when((k < 2) & (tile >= 8))
        def _():
            off = 8 * WB_A0 + (tile - 8) * WB_A1
            pltpu.sync_copy(acc_sh.at[pl.ds(off, WB_A1)],
                            out_hbm.at[pl.ds(b * S + base_k + off, WB_A1)])

        @pl.when(k == 2)
        def _():
            off = tile * OUT_TB
            pltpu.sync_copy(acc_sh.at[pl.ds(off, OUT_TB)],
                            out_hbm.at[pl.ds(b * S + base_k + off, OUT_TB)])

        plsc.subcore_barrier()
        return 0

    lax.fori_loop(0, NTASK // NC, _task, 0)


@functools.partial(
    pl.kernel,
    out_type=jax.ShapeDtypeStruct((B * S,), jnp.float32),
    mesh=plsc.VectorSubcoreMesh(core_axis_name="c", subcore_axis_name="s"),
    scratch_types=[
        pltpu.VMEM((2 * PIECE,), jnp.int32),
        pltpu.VMEM((2 * PIECE,), jnp.float32),
        pltpu.VMEM((2 * PC,), jnp.int32),
        pltpu.VMEM((2 * PIECE,), jnp.float32),
        pltpu.VMEM((ZW,), jnp.float32),
        pltpu.VMEM_SHARED((ACC,), jnp.float32),
        pltpu.SemaphoreType.DMA,
        pltpu.SemaphoreType.DMA,
        pltpu.SemaphoreType.DMA,
    ],
    compiler_params=pltpu.CompilerParams(needs_layout_passes=False),
)
def _scatter_add_kernel(feat_hbm, idx_hbm, out_hbm, idx_v, feat_v, adjc,
                        featc, zero_v, acc_sh, sem_in, sem_sc, sem_z):
    _body(feat_hbm, idx_hbm, out_hbm, idx_v, feat_v, adjc, featc, zero_v,
          acc_sh, sem_in, sem_sc, sem_z)


def kernel(features, indices):
    feat_flat = features.reshape(B * F)
    idx = indices.reshape(B * F).astype(jnp.int32)
    oh = idx // PLANE
    r = idx - oh * PLANE
    ow = r // C
    oc = r - ow * C
    ridx = oh * PLANE + oc * OW + ow
    out = _scatter_add_kernel(feat_flat, ridx)
    return out.reshape(B, OH, C, OW).transpose(0, 1, 3, 2)


# speedup vs baseline: 1.3705x; 1.3705x over previous
import functools

import jax
import jax.numpy as jnp
from jax import lax
from jax.experimental import pallas as pl
from jax.experimental.pallas import tpu as pltpu
from jax.experimental.pallas import tpu_sc as plsc

B = 8
H = W = 112
C = 96
F = H * W * C
UPS = 2
OH = H * UPS
OW = W * UPS
PLANE = OW * C
S = OH * PLANE

NC = 2
NS = 16
L = 16

NCHUNK = 3
KP = 75
CHUNKA = KP * PLANE
CHUNKB = (OH - 2 * KP) * PLANE
DUMP = 2048
ACC = CHUNKA + DUMP
NTASK = B * NCHUNK

PER_TILE = F // NS
PIECE = 2688
NPIECE = PER_TILE // PIECE
PC = PIECE + 128
ACC_Z = ACC // NS
ZW = 5312
WB_A0 = 788 * 128
WB_A1 = 787 * 128
OUT_TB = CHUNKB // NS


def _body(feat_hbm, idx_hbm, out_hbm, idx_v, feat_v, adjc, featc, zero_v,
          acc_sh, sem_in, sem_sc, sem_z):
    core = lax.axis_index("c")
    tile = lax.axis_index("s")
    iota = lax.iota(jnp.int32, L)

    def _zinit(g, _):
        zero_v[pl.ds(g * L, L)] = jnp.zeros((L,), jnp.float32)
        return 0

    lax.fori_loop(0, ZW // L, _zinit, 0)

    def _stage_start(p, par, in_base):
        src = in_base + p * PIECE
        pltpu.async_copy(idx_hbm.at[pl.ds(src, PIECE)],
                         idx_v.at[pl.ds(par * PIECE, PIECE)], sem_in)
        pltpu.async_copy(feat_hbm.at[pl.ds(src, PIECE)],
                         feat_v.at[pl.ds(par * PIECE, PIECE)], sem_in)

    def _stage_wait(p, par, in_base):
        src = in_base + p * PIECE
        pltpu.make_async_copy(idx_hbm.at[pl.ds(src, PIECE)],
                              idx_v.at[pl.ds(par * PIECE, PIECE)],
                              sem_in).wait()
        pltpu.make_async_copy(feat_hbm.at[pl.ds(src, PIECE)],
                              feat_v.at[pl.ds(par * PIECE, PIECE)],
                              sem_in).wait()

    def _compact(par, base_k, chunk_size):
        def _row(j, wptr):
            locs, oks, vs, pcs = [], [], [], []
            for g in range(128 // L):
                off = par * PIECE + j * 128 + g * L
                raw = idx_v[pl.ds(off, L)]
                local = raw - base_k
                ok = (local >= 0) & (local < chunk_size)
                locs.append(local)
                oks.append(ok)
                vs.append(feat_v[pl.ds(off, L)])
                pcs.append(plsc.all_reduce_population_count(ok)[0])
            offs = [wptr]
            for g in range(128 // L):
                offs.append(offs[g] + pcs[g])
            for g in range(128 // L):
                plsc.store_compressed(adjc.at[pl.ds(par * PC + offs[g], L)],
                                      locs[g], mask=oks[g])
                plsc.store_compressed(
                    featc.at[pl.ds(par * PIECE + offs[g], L)],
                    vs[g], mask=oks[g])
            return offs[128 // L]

        return lax.fori_loop(0, PIECE // 128, _row, jnp.int32(0))

    def _pad(par, cnt, p):
        for q in range(8):
            offs = (p * 256 + q * 16 + tile * 64) & (DUMP - 16)
            adjc[pl.ds(par * PC + cnt + q * L, L)] = CHUNKA + offs + iota

    def _scatter_fire(par, rows):
        def _row(j, _):
            pltpu.async_copy(
                featc.at[pl.ds(par * PIECE + j * 128, 128)],
                acc_sh.at[adjc.at[pl.ds(par * PC + j * 128, 128)]],
                sem_sc, add=True)
            return 0

        lax.fori_loop(0, rows, _row, 0)

    def _scatter_drain(par, rows):
        def _row(j, _):
            pltpu.make_async_copy(
                featc.at[pl.ds(par * PIECE + j * 128, 128)],
                acc_sh.at[adjc.at[pl.ds(par * PC + j * 128, 128)]],
                sem_sc).wait()
            return 0

        lax.fori_loop(0, rows, _row, 0)

    def _task(i, _):
        t = i * NC + core
        b = t // NCHUNK
        k = t % NCHUNK
        base_k = k * CHUNKA
        chunk_size = jnp.where(k == 2, CHUNKB, CHUNKA)
        in_base = b * F + tile * PER_TILE

        def _zfire(q, _):
            pltpu.async_copy(zero_v,
                             acc_sh.at[pl.ds(tile * ACC_Z + q * ZW, ZW)],
                             sem_z)
            return 0

        def _zdrain(q, _):
            pltpu.make_async_copy(
                zero_v, acc_sh.at[pl.ds(tile * ACC_Z + q * ZW, ZW)],
                sem_z).wait()
            return 0

        lax.fori_loop(0, ACC_Z // ZW, _zfire, 0)
        lax.fori_loop(0, ACC_Z // ZW, _zdrain, 0)
        plsc.subcore_barrier()

        _stage_start(0, 0, in_base)

        def _piece(p, rows_prev):
            cur = lax.rem(p, 2)
            nxt = 1 - cur
            _stage_wait(p, cur, in_base)
            cnt = _compact(cur, base_k, chunk_size)
            _pad(cur, cnt, p)
            rows = (cnt + 127) // 128

            @pl.when(p > 0)
            def _():
                _scatter_drain(nxt, rows_prev)

            _scatter_fire(cur, rows)

            @pl.when(p + 1 < NPIECE)
            def _():
                _stage_start(p + 1, nxt, in_base)

            return rows

        rows_last = lax.fori_loop(0, NPIECE, _piece, jnp.int32(0))
        _scatter_drain((NPIECE - 1) % 2, rows_last)
        plsc.subcore_barrier()

        @pl.when((k < 2) & (tile < 8))
        def _():
            off = tile * WB_A0
            pltpu.sync_copy(acc_sh.at[pl.ds(off, WB_A0)],
                            out_hbm.at[pl.ds(b * S + base_k + off, WB_A0)])

        @pl.when((k < 2) & (tile >= 8))
        def _():
            off = 8 * WB_A0 + (tile - 8) * WB_A1
            pltpu.sync_copy(acc_sh.at[pl.ds(off, WB_A1)],
                            out_hbm.at[pl.ds(b * S + base_k + off, WB_A1)])

        @pl.when(k == 2)
        def _():
            off = tile * OUT_TB
            pltpu.sync_copy(acc_sh.at[pl.ds(off, OUT_TB)],
                            out_hbm.at[pl.ds(b * S + base_k + off, OUT_TB)])

        plsc.subcore_barrier()
        return 0

    lax.fori_loop(0, NTASK // NC, _task, 0)


@functools.partial(
    pl.kernel,
    out_type=jax.ShapeDtypeStruct((B * S,), jnp.float32),
    mesh=plsc.VectorSubcoreMesh(core_axis_name="c", subcore_axis_name="s"),
    scratch_types=[
        pltpu.VMEM((2 * PIECE,), jnp.int32),
        pltpu.VMEM((2 * PIECE,), jnp.float32),
        pltpu.VMEM((2 * PC,), jnp.int32),
        pltpu.VMEM((2 * PIECE,), jnp.float32),
        pltpu.VMEM((ZW,), jnp.float32),
        pltpu.VMEM_SHARED((ACC,), jnp.float32),
        pltpu.SemaphoreType.DMA,
        pltpu.SemaphoreType.DMA,
        pltpu.SemaphoreType.DMA,
    ],
    compiler_params=pltpu.CompilerParams(needs_layout_passes=False),
)
def _scatter_add_kernel(feat_hbm, idx_hbm, out_hbm, idx_v, feat_v, adjc,
                        featc, zero_v, acc_sh, sem_in, sem_sc, sem_z):
    _body(feat_hbm, idx_hbm, out_hbm, idx_v, feat_v, adjc, featc, zero_v,
          acc_sh, sem_in, sem_sc, sem_z)


def kernel(features, indices):
    feat_flat = features.reshape(B * F)
    idx = indices.reshape(B * F).astype(jnp.int32)
    oh = idx // PLANE
    r = idx - oh * PLANE
    ow = r // C
    oc = r - ow * C
    ridx = oh * PLANE + oc * OW + ow
    out = _scatter_add_kernel(feat_flat, ridx)
    return out.reshape(B, OH, C, OW).transpose(0, 1, 3, 2)
